# baseline (speedup 1.0000x reference)
import functools

import numpy as np
import jax
import jax.numpy as jnp
from jax import lax
from jax.experimental import pallas as pl
from jax.experimental.pallas import tpu as pltpu
from jax.experimental.pallas import tpu_sc as plsc

N_NODES = 50000
N_EDGES = 800000
SH_DIM = 9
ONEHOT = 64
NBASIS = 8
LATENT = 64
MUL = 8
R_MAX = 6.0
OUT_DIM = MUL * 9
AVG_NEIGH = 16.0
EPS = 1e-8

_BE = 3200
_GRID_E = N_EDGES // _BE
_BN = 2000

_SIN_C = [3.2191201543092185, -14.883436518674236, 42.02049115694533,
          -76.70215249557859, 81.60506489900605, -41.341698212287454,
          6.283185281863447]


def _build_expand():
    ew = np.zeros((3 * MUL, OUT_DIM), np.float32)
    es = np.zeros((SH_DIM, OUT_DIM), np.float32)
    for m in range(MUL):
        ew[m, m] = 1.0
        es[0, m] = 1.0
    for m in range(MUL):
        for k in range(3):
            f = MUL + m * 3 + k
            ew[MUL + m, f] = 1.0
            es[1 + k, f] = 1.0
    for m in range(MUL):
        for k in range(5):
            f = 4 * MUL + m * 5 + k
            ew[2 * MUL + m, f] = 1.0
            es[4 + k, f] = 1.0
    return ew, es


_EW_NP, _ES_NP = _build_expand()


def _edge_body(ohT_ref, shT_ref, len_ref, w1T_ref, w2T_ref, wenvT_ref,
               ewT_ref, esT_ref, b1c_ref, b2c_ref, benvc_ref, bwc_ref,
               latT_ref, featT_ref, featR_ref, cut_ref):
    r = len_ref[0]
    arg = bwc_ref[...] * (r * (1.0 / R_MAX))
    u = arg * (1.0 / (2.0 * np.pi))
    f = u - jnp.floor(u + 0.5)
    z = f * f
    s = jnp.float32(_SIN_C[0])
    for c in _SIN_C[1:]:
        s = s * z + jnp.float32(c)
    s = s * f
    pref = (2.0 / R_MAX) ** 0.5
    besT = s * (pref / r)
    x = r * (1.0 / R_MAX)
    x2 = x * x
    x3 = x2 * x
    x6 = x3 * x3
    x7 = x6 * x
    x8 = x7 * x
    cut = 1.0 - 28.0 * x6 + 48.0 * x7 - 21.0 * x8
    cut = jnp.where(x < 1.0, cut, 0.0)
    mask = cut > 0.0
    w1T = w1T_ref[...]
    hT = (jnp.dot(w1T[:, 0:ONEHOT], ohT_ref[...],
                  preferred_element_type=jnp.float32)
          + jnp.dot(w1T[:, ONEHOT:], besT,
                    preferred_element_type=jnp.float32)
          + b1c_ref[...])
    hT = hT * jax.nn.sigmoid(hT)
    latT = jnp.dot(w2T_ref[...], hT,
                   preferred_element_type=jnp.float32) + b2c_ref[...]
    latT = jnp.where(mask, cut * latT, 0.0)
    latT_ref[...] = latT
    wtsT = jnp.dot(wenvT_ref[...], latT,
                   preferred_element_type=jnp.float32) + benvc_ref[...]
    featT = (jnp.dot(ewT_ref[...], wtsT, preferred_element_type=jnp.float32)
             * jnp.dot(esT_ref[...], shT_ref[...],
                       preferred_element_type=jnp.float32))
    featT = jnp.where(mask, featT, 0.0)
    featT_ref[...] = featT
    ft = featT.T
    featR_ref[:, 0:40] = ft[:, 0:40]
    featR_ref[:, 64:96] = ft[:, 40:OUT_DIM]
    featR_ref[:, 96:104] = jnp.zeros((_BE, 8), jnp.float32)
    cut_ref[...] = cut[None]


def _edge_call(edge_one_hot, edge_sh, edge_length, bessel_w, W1, b1, W2, b2,
               Wenv, benv):
    def full(shape):
        return pl.BlockSpec(shape, lambda i: (0, 0))

    return pl.pallas_call(
        _edge_body,
        grid=(_GRID_E,),
        in_specs=[pl.BlockSpec((ONEHOT, _BE), lambda i: (0, i)),
                  pl.BlockSpec((SH_DIM, _BE), lambda i: (0, i)),
                  pl.BlockSpec((1, 1, _BE), lambda i: (i, 0, 0)),
                  full((LATENT, ONEHOT + NBASIS)), full((LATENT, LATENT)),
                  full((3 * MUL, LATENT)), full((OUT_DIM, 3 * MUL)),
                  full((OUT_DIM, SH_DIM)), full((LATENT, 1)),
                  full((LATENT, 1)), full((3 * MUL, 1)), full((NBASIS, 1))],
        out_specs=[pl.BlockSpec((LATENT, _BE), lambda i: (0, i)),
                   pl.BlockSpec((OUT_DIM, _BE), lambda i: (0, i)),
                   pl.BlockSpec((_BE, 128), lambda i: (i, 0)),
                   pl.BlockSpec((1, 1, _BE), lambda i: (i, 0, 0))],
        out_shape=[jax.ShapeDtypeStruct((LATENT, N_EDGES), jnp.float32),
                   jax.ShapeDtypeStruct((OUT_DIM, N_EDGES), jnp.float32),
                   jax.ShapeDtypeStruct((N_EDGES, 128), jnp.float32),
                   jax.ShapeDtypeStruct((_GRID_E, 1, _BE), jnp.float32)],
    )(edge_one_hot.T, edge_sh.T, edge_length.reshape(_GRID_E, 1, _BE), W1.T,
      W2.T, Wenv.T, jnp.asarray(_EW_NP.T), jnp.asarray(_ES_NP.T),
      b1.reshape(-1, 1), b2.reshape(-1, 1), benv.reshape(-1, 1),
      bessel_w.reshape(-1, 1))


_NSUB = 16
_NCORE = 2
_E_SUB = N_EDGES // _NSUB
_K = 64
_NCHUNK = _E_SUB // _K
_W = 40
_ZROWS = 3128


def _scatter_body(feat_hbm, center_hbm, zeros_hbm, out_hbm, accum,
                  fbuf0, fbuf1, cbuf0, cbuf1, ctail, sem0, sem1, ssem0, ssem1):
    c = lax.axis_index("c")
    s = lax.axis_index("s")
    @pl.when(s < _NSUB - 1)
    def _():
        pltpu.sync_copy(zeros_hbm, accum.at[pl.ds(s * _ZROWS, _ZROWS)])

    @pl.when(s == _NSUB - 1)
    def _():
        pltpu.sync_copy(zeros_hbm.at[pl.ds(0, 3080)],
                        accum.at[pl.ds(15 * _ZROWS, 3080)])

    plsc.subcore_barrier()
    col0 = c * 64
    ebase = s * _E_SUB

    def start(j, fbuf, cbuf, sem):
        off = pl.multiple_of(ebase + j * _K, 8)
        pltpu.async_copy(feat_hbm.at[pl.ds(off, _K), pl.ds(col0, _W)],
                         fbuf, sem)
        pltpu.async_copy(center_hbm.at[pl.ds(off, _K)], cbuf, sem)

    def drain(fbuf, cbuf, sem):
        pltpu.make_async_copy(feat_hbm.at[pl.ds(0, _K), pl.ds(0, _W)],
                              fbuf, sem).wait()
        pltpu.make_async_copy(center_hbm.at[pl.ds(0, _K)], cbuf, sem).wait()

    def enq_scatter(fbuf, cbuf, ssem):
        pltpu.async_copy(fbuf, accum.at[cbuf], ssem, add=True)

    def wait_scatter(fbuf, cbuf, ssem):
        pltpu.make_async_copy(fbuf, accum.at[cbuf], ssem).wait()

    start(0, fbuf0, cbuf0, sem0)
    start(1, fbuf1, cbuf1, sem1)

    def pair(m, carry):
        j = m * 2
        drain(fbuf0, cbuf0, sem0)
        enq_scatter(fbuf0, cbuf0, ssem0)
        drain(fbuf1, cbuf1, sem1)
        enq_scatter(fbuf1, cbuf1, ssem1)
        wait_scatter(fbuf0, cbuf0, ssem0)
        start(j + 2, fbuf0, cbuf0, sem0)
        wait_scatter(fbuf1, cbuf1, ssem1)
        start(j + 3, fbuf1, cbuf1, sem1)
        return carry

    lax.fori_loop(0, (_NCHUNK - 3) // 2, pair, 0)
    drain(fbuf0, cbuf0, sem0)
    enq_scatter(fbuf0, cbuf0, ssem0)
    drain(fbuf1, cbuf1, sem1)
    enq_scatter(fbuf1, cbuf1, ssem1)
    wait_scatter(fbuf0, cbuf0, ssem0)
    start(_NCHUNK - 1, fbuf0, cbuf0, sem0)
    wait_scatter(fbuf1, cbuf1, ssem1)
    drain(fbuf0, cbuf0, sem0)
    pltpu.sync_copy(fbuf0, accum.at[cbuf0], add=True)
    toff = pl.multiple_of(ebase + _NCHUNK * _K, 8)
    pltpu.sync_copy(feat_hbm.at[pl.ds(toff, 16), pl.ds(col0, _W)],
                    fbuf0.at[pl.ds(0, 16)])
    pltpu.sync_copy(center_hbm.at[pl.ds(toff, 16)], ctail)
    pltpu.sync_copy(fbuf0.at[pl.ds(0, 16)], accum.at[ctail], add=True)
    plsc.subcore_barrier()
    def copy_out(row0, nrows):
        @pl.when(c == 0)
        def _():
            pltpu.sync_copy(accum.at[pl.ds(row0, nrows)],
                            out_hbm.at[pl.ds(row0, nrows), pl.ds(0, _W)])

        @pl.when(c == 1)
        def _():
            pltpu.sync_copy(accum.at[pl.ds(row0, nrows), pl.ds(0, 32)],
                            out_hbm.at[pl.ds(row0, nrows), pl.ds(_W, 32)])

    @pl.when(s < _NSUB - 1)
    def _():
        copy_out(s * 3128, 3128)

    @pl.when(s == _NSUB - 1)
    def _():
        copy_out(15 * 3128, 3080)


@functools.lru_cache(maxsize=1)
def _get_scatter():
    return pl.kernel(
        _scatter_body,
        out_type=jax.ShapeDtypeStruct((N_NODES, OUT_DIM), jnp.float32),
        mesh=plsc.VectorSubcoreMesh(core_axis_name="c", subcore_axis_name="s",
                                    num_cores=_NCORE, num_subcores=_NSUB),
        scratch_types=[
            pltpu.VMEM_SHARED((N_NODES, _W), jnp.float32),
            pltpu.VMEM((_K, _W), jnp.float32),
            pltpu.VMEM((_K, _W), jnp.float32),
            pltpu.VMEM((_K,), jnp.int32),
            pltpu.VMEM((_K,), jnp.int32),
            pltpu.VMEM((16,), jnp.int32),
            pltpu.SemaphoreType.DMA,
            pltpu.SemaphoreType.DMA,
            pltpu.SemaphoreType.DMA,
            pltpu.SemaphoreType.DMA,
        ],
        compiler_params=pltpu.CompilerParams(use_tc_tiling_on_sc=False),
    )


def _ln_body(x_ref, g0_ref, bt0_ref, g1_ref, g2_ref, o_ref):
    x = x_ref[...] * jnp.float32(AVG_NEIGH ** -0.5)
    sc = x[:, 0:MUL]
    mu = jnp.mean(sc, axis=1, keepdims=True)
    var = jnp.mean((sc - mu) ** 2, axis=1, keepdims=True)
    sn = (sc - mu) * lax.rsqrt(var + EPS) * g0_ref[...] + bt0_ref[...]
    v1 = x[:, MUL:4 * MUL]
    n1 = lax.rsqrt(jnp.mean(v1 * v1, axis=1, keepdims=True) + EPS)
    v1 = v1 * n1 * g1_ref[...]
    v2 = x[:, 4 * MUL:9 * MUL]
    n2 = lax.rsqrt(jnp.mean(v2 * v2, axis=1, keepdims=True) + EPS)
    v2 = v2 * n2 * g2_ref[...]
    o_ref[...] = jnp.concatenate([sn, v1, v2], axis=1)


def _ln_call(nodes, g0, bt0, g1, g2):
    grid = (N_NODES // _BN,)

    def full(shape):
        return pl.BlockSpec(shape, lambda i: (0, 0))

    return pl.pallas_call(
        _ln_body,
        grid=grid,
        in_specs=[pl.BlockSpec((_BN, OUT_DIM), lambda i: (i, 0)),
                  full((1, MUL)), full((1, MUL)),
                  full((1, 3 * MUL)), full((1, 5 * MUL))],
        out_specs=pl.BlockSpec((_BN, OUT_DIM), lambda i: (i, 0)),
        out_shape=jax.ShapeDtypeStruct((N_NODES, OUT_DIM), jnp.float32),
    )(nodes, g0.reshape(1, -1), bt0.reshape(1, -1),
      jnp.repeat(g1, 3).reshape(1, -1), jnp.repeat(g2, 5).reshape(1, -1))


def kernel(edge_index, atom_type, bond_type, edge_sh, edge_length, edge_one_hot,
           bessel_w, W1, b1, W2, b2, Wenv, benv, g0, bt0, g1, g2):
    latT, featT, featR, cut2 = _edge_call(edge_one_hot, edge_sh, edge_length,
                                          bessel_w, W1, b1, W2, b2, Wenv, benv)
    lat = latT.T
    feat = featT.T
    zeros = jnp.zeros((_ZROWS, _W), jnp.float32)
    nodes = _get_scatter()(featR, edge_index[0], zeros)
    node_out = _ln_call(nodes, g0, bt0, g1, g2)
    cutoff = cut2.reshape(N_EDGES)
    mask = cutoff > 0.0
    return lat, node_out, feat, cutoff, mask


# speedup vs baseline: 1.1694x; 1.1694x over previous
import functools

import numpy as np
import jax
import jax.numpy as jnp
from jax import lax
from jax.experimental import pallas as pl
from jax.experimental.pallas import tpu as pltpu
from jax.experimental.pallas import tpu_sc as plsc

N_NODES = 50000
N_EDGES = 800000
SH_DIM = 9
ONEHOT = 64
NBASIS = 8
LATENT = 64
MUL = 8
R_MAX = 6.0
OUT_DIM = MUL * 9
AVG_NEIGH = 16.0
EPS = 1e-8

_BE = 3200
_GRID_E = N_EDGES // _BE
_BN = 2000

_SIN_C = [3.2191201543092185, -14.883436518674236, 42.02049115694533,
          -76.70215249557859, 81.60506489900605, -41.341698212287454,
          6.283185281863447]


def _build_expand():
    ew = np.zeros((3 * MUL, OUT_DIM), np.float32)
    es = np.zeros((SH_DIM, OUT_DIM), np.float32)
    for m in range(MUL):
        ew[m, m] = 1.0
        es[0, m] = 1.0
    for m in range(MUL):
        for k in range(3):
            f = MUL + m * 3 + k
            ew[MUL + m, f] = 1.0
            es[1 + k, f] = 1.0
    for m in range(MUL):
        for k in range(5):
            f = 4 * MUL + m * 5 + k
            ew[2 * MUL + m, f] = 1.0
            es[4 + k, f] = 1.0
    return ew, es


_EW_NP, _ES_NP = _build_expand()


def _edge_body(ohT_ref, shT_ref, len_ref, w1T_ref, w2T_ref, wenvT_ref,
               ewT_ref, esT_ref, b1c_ref, b2c_ref, benvc_ref, bwc_ref,
               latT_ref, featT_ref, featR_ref, cut_ref):
    r = len_ref[0]
    arg = bwc_ref[...] * (r * (1.0 / R_MAX))
    u = arg * (1.0 / (2.0 * np.pi))
    f = u - jnp.floor(u + 0.5)
    z = f * f
    s = jnp.float32(_SIN_C[0])
    for c in _SIN_C[1:]:
        s = s * z + jnp.float32(c)
    s = s * f
    pref = (2.0 / R_MAX) ** 0.5
    besT = s * (pref / r)
    x = r * (1.0 / R_MAX)
    x2 = x * x
    x3 = x2 * x
    x6 = x3 * x3
    x7 = x6 * x
    x8 = x7 * x
    cut = 1.0 - 28.0 * x6 + 48.0 * x7 - 21.0 * x8
    cut = jnp.where(x < 1.0, cut, 0.0)
    mask = cut > 0.0
    w1T = w1T_ref[...]
    hT = (jnp.dot(w1T[:, 0:ONEHOT], ohT_ref[...],
                  preferred_element_type=jnp.float32)
          + jnp.dot(w1T[:, ONEHOT:], besT,
                    preferred_element_type=jnp.float32)
          + b1c_ref[...])
    hT = hT * jax.nn.sigmoid(hT)
    latT = jnp.dot(w2T_ref[...], hT,
                   preferred_element_type=jnp.float32) + b2c_ref[...]
    latT = jnp.where(mask, cut * latT, 0.0)
    latT_ref[...] = latT
    wtsT = jnp.dot(wenvT_ref[...], latT,
                   preferred_element_type=jnp.float32) + benvc_ref[...]
    featT = (jnp.dot(ewT_ref[...], wtsT, preferred_element_type=jnp.float32)
             * jnp.dot(esT_ref[...], shT_ref[...],
                       preferred_element_type=jnp.float32))
    featT = jnp.where(mask, featT, 0.0)
    featT_ref[...] = featT
    featR_ref[:, 0:OUT_DIM] = featT.T
    cut_ref[...] = cut[None]


def _edge_call(edge_one_hot, edge_sh, edge_length, bessel_w, W1, b1, W2, b2,
               Wenv, benv):
    def full(shape):
        return pl.BlockSpec(shape, lambda i: (0, 0))

    return pl.pallas_call(
        _edge_body,
        grid=(_GRID_E,),
        in_specs=[pl.BlockSpec((ONEHOT, _BE), lambda i: (0, i)),
                  pl.BlockSpec((SH_DIM, _BE), lambda i: (0, i)),
                  pl.BlockSpec((1, 1, _BE), lambda i: (i, 0, 0)),
                  full((LATENT, ONEHOT + NBASIS)), full((LATENT, LATENT)),
                  full((3 * MUL, LATENT)), full((OUT_DIM, 3 * MUL)),
                  full((OUT_DIM, SH_DIM)), full((LATENT, 1)),
                  full((LATENT, 1)), full((3 * MUL, 1)), full((NBASIS, 1))],
        out_specs=[pl.BlockSpec((LATENT, _BE), lambda i: (0, i)),
                   pl.BlockSpec((OUT_DIM, _BE), lambda i: (0, i)),
                   pl.BlockSpec((_BE, 128), lambda i: (i, 0)),
                   pl.BlockSpec((1, 1, _BE), lambda i: (i, 0, 0))],
        out_shape=[jax.ShapeDtypeStruct((LATENT, N_EDGES), jnp.float32),
                   jax.ShapeDtypeStruct((OUT_DIM, N_EDGES), jnp.float32),
                   jax.ShapeDtypeStruct((N_EDGES, 128), jnp.float32),
                   jax.ShapeDtypeStruct((_GRID_E, 1, _BE), jnp.float32)],
    )(edge_one_hot.T, edge_sh.T, edge_length.reshape(_GRID_E, 1, _BE), W1.T,
      W2.T, Wenv.T, jnp.asarray(_EW_NP.T), jnp.asarray(_ES_NP.T),
      b1.reshape(-1, 1), b2.reshape(-1, 1), benv.reshape(-1, 1),
      bessel_w.reshape(-1, 1))


_NSUB = 16
_NCORE = 2
_E_SUB = N_EDGES // _NSUB
_K = 112
_NCHUNK = _E_SUB // _K
_HALF = N_NODES // _NCORE
_ZROWS = 1568
_ACC_ROWS = _ZROWS * _NSUB


def _scatter_body(feat_hbm, center_hbm, zeros_hbm, out_hbm, accum,
                  fbuf0, fbuf1, cbuf0, cbuf1, libuf0, libuf1, ltail,
                  sem0, sem1):
    c = lax.axis_index("c")
    s = lax.axis_index("s")
    pltpu.sync_copy(zeros_hbm, accum.at[pl.ds(s * _ZROWS, _ZROWS)])
    plsc.subcore_barrier()
    base_node = c * _HALF
    ebase = s * _E_SUB

    def start(j, fbuf, cbuf, sem):
        off = pl.multiple_of(ebase + j * _K, 8)
        pltpu.async_copy(feat_hbm.at[pl.ds(off, _K), pl.ds(0, OUT_DIM)],
                         fbuf, sem)
        pltpu.async_copy(center_hbm.at[pl.ds(off, _K)], cbuf, sem)

    def drain(fbuf, cbuf, sem):
        pltpu.make_async_copy(feat_hbm.at[pl.ds(0, _K), pl.ds(0, OUT_DIM)],
                              fbuf, sem).wait()
        pltpu.make_async_copy(center_hbm.at[pl.ds(0, _K)], cbuf, sem).wait()

    def scatter(fbuf, cbuf, libuf):
        for t in range(_K // 16):
            ci = cbuf[pl.ds(t * 16, 16)]
            li = ci - base_node
            oob = (li < 0) | (li >= _HALF)
            libuf[pl.ds(t * 16, 16)] = jnp.where(oob, _HALF + (ci & 63), li)
        pltpu.sync_copy(fbuf, accum.at[libuf], add=True)

    start(0, fbuf0, cbuf0, sem0)

    def pair(m, carry):
        j = m * 2
        start(j + 1, fbuf1, cbuf1, sem1)
        drain(fbuf0, cbuf0, sem0)
        scatter(fbuf0, cbuf0, libuf0)
        start(j + 2, fbuf0, cbuf0, sem0)
        drain(fbuf1, cbuf1, sem1)
        scatter(fbuf1, cbuf1, libuf1)
        return carry

    lax.fori_loop(0, (_NCHUNK - 1) // 2, pair, 0)
    start(_NCHUNK - 1, fbuf1, cbuf1, sem1)
    drain(fbuf0, cbuf0, sem0)
    scatter(fbuf0, cbuf0, libuf0)
    drain(fbuf1, cbuf1, sem1)
    scatter(fbuf1, cbuf1, libuf1)
    toff = pl.multiple_of(ebase + _NCHUNK * _K, 8)
    pltpu.sync_copy(feat_hbm.at[pl.ds(toff, 48), pl.ds(0, OUT_DIM)],
                    fbuf0.at[pl.ds(0, 48)])
    pltpu.sync_copy(center_hbm.at[pl.ds(toff, 48)], cbuf0.at[pl.ds(0, 48)])
    for t in range(3):
        ci = cbuf0[pl.ds(t * 16, 16)]
        li = ci - base_node
        oob = (li < 0) | (li >= _HALF)
        ltail[pl.ds(t * 16, 16)] = jnp.where(oob, _HALF + (ci & 63), li)
    pltpu.sync_copy(fbuf0.at[pl.ds(0, 48)], accum.at[ltail], add=True)
    plsc.subcore_barrier()
    off = s * 1560
    pltpu.sync_copy(accum.at[pl.ds(off, 1560)],
                    out_hbm.at[pl.ds(c * _HALF + off, 1560)])

    @pl.when(s == _NSUB - 1)
    def _():
        pltpu.sync_copy(accum.at[pl.ds(16 * 1560, 40)],
                        out_hbm.at[pl.ds(c * _HALF + 16 * 1560, 40)])


@functools.lru_cache(maxsize=1)
def _get_scatter():
    return pl.kernel(
        _scatter_body,
        out_type=jax.ShapeDtypeStruct((N_NODES, OUT_DIM), jnp.float32),
        mesh=plsc.VectorSubcoreMesh(core_axis_name="c", subcore_axis_name="s",
                                    num_cores=_NCORE, num_subcores=_NSUB),
        scratch_types=[
            pltpu.VMEM_SHARED((_ACC_ROWS, OUT_DIM), jnp.float32),
            pltpu.VMEM((_K, OUT_DIM), jnp.float32),
            pltpu.VMEM((_K, OUT_DIM), jnp.float32),
            pltpu.VMEM((_K,), jnp.int32),
            pltpu.VMEM((_K,), jnp.int32),
            pltpu.VMEM((_K,), jnp.int32),
            pltpu.VMEM((_K,), jnp.int32),
            pltpu.VMEM((48,), jnp.int32),
            pltpu.SemaphoreType.DMA,
            pltpu.SemaphoreType.DMA,
        ],
        compiler_params=pltpu.CompilerParams(use_tc_tiling_on_sc=False),
    )


def _ln_body(x_ref, g0_ref, bt0_ref, g1_ref, g2_ref, o_ref):
    x = x_ref[...] * jnp.float32(AVG_NEIGH ** -0.5)
    sc = x[:, 0:MUL]
    mu = jnp.mean(sc, axis=1, keepdims=True)
    var = jnp.mean((sc - mu) ** 2, axis=1, keepdims=True)
    sn = (sc - mu) * lax.rsqrt(var + EPS) * g0_ref[...] + bt0_ref[...]
    v1 = x[:, MUL:4 * MUL]
    n1 = lax.rsqrt(jnp.mean(v1 * v1, axis=1, keepdims=True) + EPS)
    v1 = v1 * n1 * g1_ref[...]
    v2 = x[:, 4 * MUL:9 * MUL]
    n2 = lax.rsqrt(jnp.mean(v2 * v2, axis=1, keepdims=True) + EPS)
    v2 = v2 * n2 * g2_ref[...]
    o_ref[...] = jnp.concatenate([sn, v1, v2], axis=1)


def _ln_call(nodes, g0, bt0, g1, g2):
    grid = (N_NODES // _BN,)

    def full(shape):
        return pl.BlockSpec(shape, lambda i: (0, 0))

    return pl.pallas_call(
        _ln_body,
        grid=grid,
        in_specs=[pl.BlockSpec((_BN, OUT_DIM), lambda i: (i, 0)),
                  full((1, MUL)), full((1, MUL)),
                  full((1, 3 * MUL)), full((1, 5 * MUL))],
        out_specs=pl.BlockSpec((_BN, OUT_DIM), lambda i: (i, 0)),
        out_shape=jax.ShapeDtypeStruct((N_NODES, OUT_DIM), jnp.float32),
    )(nodes, g0.reshape(1, -1), bt0.reshape(1, -1),
      jnp.repeat(g1, 3).reshape(1, -1), jnp.repeat(g2, 5).reshape(1, -1))


def kernel(edge_index, atom_type, bond_type, edge_sh, edge_length, edge_one_hot,
           bessel_w, W1, b1, W2, b2, Wenv, benv, g0, bt0, g1, g2):
    latT, featT, featR, cut2 = _edge_call(edge_one_hot, edge_sh, edge_length,
                                          bessel_w, W1, b1, W2, b2, Wenv, benv)
    lat = latT.T
    feat = featT.T
    zeros = jnp.zeros((_ZROWS, OUT_DIM), jnp.float32)
    nodes = _get_scatter()(featR, edge_index[0], zeros)
    node_out = _ln_call(nodes, g0, bt0, g1, g2)
    cutoff = cut2.reshape(N_EDGES)
    mask = cutoff > 0.0
    return lat, node_out, feat, cutoff, mask
